# baseline (speedup 1.0000x reference)
import functools

import jax
import jax.numpy as jnp
from jax import lax
from jax.experimental import pallas as pl
from jax.experimental.pallas import tpu as pltpu
from jax.experimental.pallas import tpu_sc as plsc

_BATCH = 4096
_SEQ = 50
_D = 128
_NC = 2
_NS = 16
_NW = _NC * _NS
_BT = _BATCH // _NW
_NBUF = 5
_NSTEP = _SEQ // _NBUF

_mesh = plsc.VectorSubcoreMesh(core_axis_name="c", subcore_axis_name="s")


@functools.partial(
    pl.kernel,
    mesh=_mesh,
    out_type=jax.ShapeDtypeStruct((_SEQ, _BATCH, _D), jnp.float32),
    scratch_types=[
        pltpu.VMEM((_SEQ, _BT), jnp.int32),
        pltpu.VMEM((_NBUF, _BT, _D), jnp.float32),
    ] + [pltpu.SemaphoreType.DMA] * (2 * _NBUF),
)
def _gather_kernel(idx_hbm, table_hbm, out_hbm, idx_v, rows_v, *sems):
    gsem = sems[:_NBUF]
    ssem = sems[_NBUF:]
    wid = lax.axis_index("s") * _NC + lax.axis_index("c")
    bbase = wid * _BT
    pltpu.sync_copy(idx_hbm.at[:, pl.ds(bbase, _BT)], idx_v)

    def g_start(s, b):
        pltpu.async_copy(table_hbm.at[idx_v.at[s]], rows_v.at[b], gsem[b])

    def g_wait(s, b):
        pltpu.make_async_copy(table_hbm.at[idx_v.at[s]], rows_v.at[b],
                              gsem[b]).wait()

    def s_start(s, b):
        pltpu.async_copy(rows_v.at[b], out_hbm.at[s, pl.ds(bbase, _BT)],
                         ssem[b])

    def s_wait(s, b):
        pltpu.make_async_copy(rows_v.at[b], out_hbm.at[s, pl.ds(bbase, _BT)],
                              ssem[b]).wait()

    for b in range(_NBUF):
        g_start(b, b)

    def step(t, _):
        for b in range(_NBUF):
            s = t * _NBUF + b
            g_wait(s, b)
            s_start(s, b)
        for b in range(_NBUF):
            s = t * _NBUF + b
            s_wait(s, b)

            @pl.when(t < _NSTEP - 1)
            def _():
                g_start(s + _NBUF, b)

        return 0

    lax.fori_loop(0, _NSTEP, step, 0)


def kernel(x, table):
    out = _gather_kernel(jnp.transpose(x), table)
    return jnp.transpose(out, (1, 0, 2))


# speedup vs baseline: 1.0020x; 1.0020x over previous
import functools

import jax
import jax.numpy as jnp
from jax import lax
from jax.experimental import pallas as pl
from jax.experimental.pallas import tpu as pltpu
from jax.experimental.pallas import tpu_sc as plsc

_BATCH = 4096
_SEQ = 50
_D = 128
_NC = 2
_NS = 16
_NW = _NC * _NS
_BT = _BATCH // _NW
_NBUF = 5
_NSTEP = _SEQ // _NBUF

_mesh = plsc.VectorSubcoreMesh(core_axis_name="c", subcore_axis_name="s")


@functools.partial(
    pl.kernel,
    mesh=_mesh,
    out_type=jax.ShapeDtypeStruct((_SEQ, _BATCH, _D), jnp.float32),
    scratch_types=[
        pltpu.VMEM((_SEQ, _BT), jnp.int32),
        pltpu.VMEM((_NBUF, _BT, _D), jnp.float32),
    ] + [pltpu.SemaphoreType.DMA] * (2 * _NBUF),
    compiler_params=pltpu.CompilerParams(skip_device_barrier=True),
)
def _gather_kernel(idx_hbm, table_hbm, out_hbm, idx_v, rows_v, *sems):
    gsem = sems[:_NBUF]
    ssem = sems[_NBUF:]
    wid = lax.axis_index("s") * _NC + lax.axis_index("c")
    bbase = wid * _BT
    pltpu.sync_copy(idx_hbm.at[:, pl.ds(bbase, _BT)], idx_v)

    def g_start(s, b):
        pltpu.async_copy(table_hbm.at[idx_v.at[s]], rows_v.at[b], gsem[b])

    def g_wait(s, b):
        pltpu.make_async_copy(table_hbm.at[idx_v.at[s]], rows_v.at[b],
                              gsem[b]).wait()

    def s_start(s, b):
        pltpu.async_copy(rows_v.at[b], out_hbm.at[s, pl.ds(bbase, _BT)],
                         ssem[b])

    def s_wait(s, b):
        pltpu.make_async_copy(rows_v.at[b], out_hbm.at[s, pl.ds(bbase, _BT)],
                              ssem[b]).wait()

    for b in range(_NBUF):
        g_start(b, b)

    def step(t, _):
        for b in range(_NBUF):
            s = t * _NBUF + b
            g_wait(s, b)
            s_start(s, b)
        for b in range(_NBUF):
            s = t * _NBUF + b
            s_wait(s, b)

            @pl.when(t < _NSTEP - 1)
            def _():
                g_start(s + _NBUF, b)

        return 0

    lax.fori_loop(0, _NSTEP, step, 0)


def kernel(x, table):
    out = _gather_kernel(jnp.transpose(x), table)
    return jnp.transpose(out, (1, 0, 2))


# speedup vs baseline: 1.0095x; 1.0075x over previous
import functools

import jax
import jax.numpy as jnp
from jax import lax
from jax.experimental import pallas as pl
from jax.experimental.pallas import tpu as pltpu
from jax.experimental.pallas import tpu_sc as plsc

_BATCH = 4096
_SEQ = 50
_D = 128
_NC = 2
_NS = 16
_NW = _NC * _NS
_BT = _BATCH // _NW
_NBUF = 5
_NSTEP = _SEQ // _NBUF

_mesh = plsc.VectorSubcoreMesh(core_axis_name="c", subcore_axis_name="s")


@functools.partial(
    pl.kernel,
    mesh=_mesh,
    out_type=jax.ShapeDtypeStruct((_SEQ, _BATCH, _D), jnp.float32),
    scratch_types=[
        pltpu.VMEM((_SEQ, _BT), jnp.int32),
        pltpu.VMEM((_NBUF, _BT, _D), jnp.float32),
    ] + [pltpu.SemaphoreType.DMA] * (2 * _NBUF),
)
def _gather_kernel(idx_hbm, table_hbm, out_hbm, idx_v, rows_v, *sems):
    gsem = sems[:_NBUF]
    ssem = sems[_NBUF:]
    wid = lax.axis_index("s") * _NC + lax.axis_index("c")
    bbase = wid * _BT
    pltpu.sync_copy(idx_hbm.at[:, pl.ds(bbase, _BT)], idx_v)

    def g_start(s, b):
        pltpu.async_copy(table_hbm.at[idx_v.at[s]], rows_v.at[b], gsem[b])

    def g_wait(s, b):
        pltpu.make_async_copy(table_hbm.at[idx_v.at[s]], rows_v.at[b],
                              gsem[b]).wait()

    def s_start(s, b):
        pltpu.async_copy(rows_v.at[b], out_hbm.at[s, pl.ds(bbase, _BT)],
                         ssem[b])

    def s_wait(s, b):
        pltpu.make_async_copy(rows_v.at[b], out_hbm.at[s, pl.ds(bbase, _BT)],
                              ssem[b]).wait()

    for b in range(_NBUF):
        g_start(b, b)

    def step(t, _):
        for b in range(_NBUF):
            s = t * _NBUF + b
            g_wait(s, b)
            s_start(s, b)
        for b in range(_NBUF):
            s = t * _NBUF + b
            s_wait(s, b)

            @pl.when(t < _NSTEP - 1)
            def _():
                g_start(s + _NBUF, b)

        return 0

    lax.fori_loop(0, _NSTEP, step, 0)


def kernel(x, table):
    out = _gather_kernel(jnp.transpose(x), table)
    return jnp.transpose(out, (1, 0, 2))
